# baseline (speedup 1.0000x reference)
import functools

import jax
import jax.numpy as jnp
import numpy as np
from jax import lax
from jax.experimental import pallas as pl
from jax.experimental.pallas import tpu as pltpu
from jax.experimental.pallas import tpu_sc as plsc

_B, _N = 256, 64
_D = 256
_EF = 64
_RUP = 5.0

_MEANS = np.linspace(0.0, _RUP, _EF, dtype=np.float32)
_DELTA = np.float32(_MEANS[1] - _MEANS[0])
_COEFF = float(np.float32(0.5) / (_DELTA * _DELTA))


_NW = 32
_TOTAL = _B * _N
_ROWS_W = _TOTAL // _NW
_CH = 128
_NCH = _ROWS_W // _CH


@functools.lru_cache(maxsize=1)
def _make_sc_gather():
    info = plsc.get_sparse_core_info()
    nc, ns = info.num_cores, info.num_subcores
    nw = nc * ns
    assert nw == _NW
    rows_w, ch, nch = _ROWS_W, _CH, _NCH

    mesh = plsc.VectorSubcoreMesh(core_axis_name="c", subcore_axis_name="s")

    @functools.partial(
        pl.kernel,
        mesh=mesh,
        out_type=jax.ShapeDtypeStruct((_TOTAL, _D), jnp.float32),
        scratch_types=[
            pltpu.VMEM((nch, ch), jnp.int32),
            pltpu.VMEM((ch, _D), jnp.float32),
            pltpu.VMEM((ch, _D), jnp.float32),
            pltpu.SemaphoreType.DMA,
            pltpu.SemaphoreType.DMA,
        ],
    )
    def sc_gather(table_hbm, idx_hbm, out_hbm, idx_v, buf0, buf1, sem0, sem1):
        wid = lax.axis_index("s") * nc + lax.axis_index("c")
        base = wid * rows_w
        pltpu.sync_copy(idx_hbm.at[wid], idx_v)
        bufs = (buf0, buf1)
        sems = (sem0, sem1)
        copies = [None, None]
        copies[0] = pltpu.async_copy(table_hbm.at[idx_v.at[0]], bufs[0], sems[0])
        for c in range(nch):
            nxt = c + 1
            if nxt < nch:
                copies[nxt % 2] = pltpu.async_copy(
                    table_hbm.at[idx_v.at[nxt]], bufs[nxt % 2], sems[nxt % 2]
                )
            copies[c % 2].wait()
            pltpu.sync_copy(bufs[c % 2], out_hbm.at[pl.ds(base + c * ch, ch)])

    return sc_gather


def _ef_body(pos_ref, eo_ref, means_ref, out_ref):
    p = pos_ref[0]
    eo = eo_ref[0]
    half = _N // 2

    sq_e = jnp.zeros((_N, half), jnp.float32)
    sq_o = jnp.zeros((_N, half), jnp.float32)
    for d in range(3):
        col = p[:, d : d + 1]
        d_e = col - eo[2 * d : 2 * d + 1, :]
        d_o = col - eo[2 * d + 1 : 2 * d + 2, :]
        sq_e = sq_e + d_e * d_e
        sq_o = sq_o + d_o * d_o

    el_e = jnp.sqrt(sq_e)
    el_o = jnp.sqrt(sq_o)

    el_e3 = jnp.broadcast_to(el_e[:, :, None], (_N, half, 2 * _EF))
    el_o3 = jnp.broadcast_to(el_o[:, :, None], (_N, half, 2 * _EF))
    lane = lax.broadcasted_iota(jnp.int32, (_N, half, 2 * _EF), 2)
    el3 = jnp.where(lane < _EF, el_e3, el_o3)

    m3 = means_ref[0][None, :, :]
    t = el3 - jnp.broadcast_to(m3, (_N, half, 2 * _EF))
    out_ref[0] = jnp.exp(t * t * (-_COEFF))


def _make_ef_call():
    half = _N // 2
    grid = (_B,)
    return pl.pallas_call(
        _ef_body,
        grid=grid,
        in_specs=[
            pl.BlockSpec((1, _N, 3), lambda b: (b, 0, 0)),
            pl.BlockSpec((1, 6, half), lambda b: (b, 0, 0)),
            pl.BlockSpec((1, 1, 2 * _EF), lambda b: (0, 0, 0)),
        ],
        out_specs=pl.BlockSpec((1, _N, half, 2 * _EF), lambda b: (b, 0, 0, 0)),
        out_shape=jax.ShapeDtypeStruct((_B, _N, half, 2 * _EF), jnp.float32),
    )


_EF_CALL = _make_ef_call()


def kernel(z, pos, z_emb):
    table = z_emb.at[0].set(0.0)
    z_flat3 = z.astype(jnp.int32).reshape(_NW, _NCH, _CH)

    pos_eo = (
        pos.reshape(_B, _N // 2, 2, 3)
        .transpose(0, 3, 2, 1)
        .reshape(_B, 6, _N // 2)
    )
    means2 = jnp.tile(jnp.asarray(_MEANS), 2).reshape(1, 1, 2 * _EF)

    emb_flat = _make_sc_gather()(table, z_flat3)
    ef4 = _EF_CALL(pos, pos_eo, means2)

    emb1 = emb_flat.reshape(_B, _N, _D)
    ef = ef4.reshape(_B, _N, _N, _EF)
    return (emb1, ef)
